# baseline (speedup 1.0000x reference)
import numpy as np
import jax
import jax.numpy as jnp
from jax import lax
from jax.experimental import pallas as pl
from jax.experimental.pallas import tpu as pltpu
from jax.experimental.pallas import tpu_sc as plsc

LANES = 16
ROWS_PER_UNIT = 192
COLS = 384
PIXELS = 147456.0
VECS_PER_ROW = COLS // LANES
NW = 32

_step = (1.0 - 0.0) / 25
_NODES128 = np.full((3328,), 2.0, dtype=np.float32)
_NODES128[np.arange(26) * 128] = np.array(
    [0.0 + _step * n for n in range(25)] + [2.0], dtype=np.float32)


def _sc_hist_kernel(inp_ref, tgt_ref, nodes_ref, out_ref,
                    buf, nodes_v, hist1d, hist2):
    c = lax.axis_index("c")
    s = lax.axis_index("s")
    wid = s * 2 + c
    tensor = wid // 16
    rem = wid - tensor * 16
    half = rem // 8
    batch = rem - half * 8
    row0 = batch * 384 + half * ROWS_PER_UNIT

    pltpu.sync_copy(nodes_ref, nodes_v)

    @pl.when(tensor == 0)
    def _():
        pltpu.sync_copy(
            inp_ref.at[pl.ds(row0 * COLS, ROWS_PER_UNIT * COLS)], buf)

    @pl.when(tensor == 1)
    def _():
        pltpu.sync_copy(
            tgt_ref.at[pl.ds(row0 * COLS, ROWS_PER_UNIT * COLS)], buf)

    zeros_i = jnp.zeros((LANES,), jnp.int32)
    ones_i = jnp.ones((LANES,), jnp.int32)
    v25f = jnp.full((LANES,), 25.0, jnp.float32)
    eps = jnp.full((LANES,), 1e-5, jnp.float32)
    lane = lax.iota(jnp.int32, LANES)
    lane_m128 = lane - 128

    for k in range(26):
        hist1d[pl.ds(k * 128, LANES)] = zeros_i
    for k in range(32):
        hist2[k, :] = zeros_i

    @plsc.parallel_loop(0, ROWS_PER_UNIT)
    def _loop(r):
        base = r * COLS
        for cvec in range(VECS_PER_ROW):
            x = buf[pl.ds(base + cvec * LANES, LANES)]
            j128 = (x * v25f + eps).astype(jnp.int32) << 7
            lo = plsc.load_gather(nodes_v, [j128])
            idx = j128 + jnp.where(x < lo, lane_m128, lane)
            plsc.addupdate_scatter(hist1d, [idx], ones_i)

    for k in range(26):
        hist2[k, :] = hist1d[pl.ds(k * 128, LANES)]
    pltpu.sync_copy(hist2, out_ref.at[wid])


def _tc_loss_kernel(h_ref, o_ref):
    h = h_ref[...].astype(jnp.float32)
    h2 = jnp.sum(h, axis=2)
    cin = h2[0:8] + h2[8:16]
    ctg = h2[16:24] + h2[24:32]
    d = jnp.sum(jnp.abs(cin - ctg), axis=0, keepdims=True)
    a = jnp.sum(cin, axis=0, keepdims=True)
    c = jnp.sum(ctg, axis=0, keepdims=True)
    li = jnp.clip(d - 0.5 * jnp.minimum(a, c), 0.0, PIXELS) / PIXELS
    mask = lax.broadcasted_iota(jnp.int32, (1, 32), 1) < 24
    li = jnp.where(mask, li, 0.0)
    o_ref[...] = jnp.sum(li, axis=(0, 1), keepdims=True) / 24.0


def kernel(_input, _target):
    inp2 = jnp.reshape(_input, (-1,))
    tgt2 = jnp.reshape(_target, (-1,))
    nodes = jnp.asarray(_NODES128)

    mesh = plsc.VectorSubcoreMesh(core_axis_name="c", subcore_axis_name="s")
    hist = pl.kernel(
        _sc_hist_kernel,
        out_type=jax.ShapeDtypeStruct((NW, 32, LANES), jnp.int32),
        mesh=mesh,
        scratch_types=[
            pltpu.VMEM((ROWS_PER_UNIT * COLS,), jnp.float32),
            pltpu.VMEM((3328,), jnp.float32),
            pltpu.VMEM((3328,), jnp.int32),
            pltpu.VMEM((32, LANES), jnp.int32),
        ],
        compiler_params=pltpu.CompilerParams(needs_layout_passes=False),
    )(inp2, tgt2, nodes)

    loss = pl.pallas_call(
        _tc_loss_kernel,
        out_shape=jax.ShapeDtypeStruct((1, 1), jnp.float32),
    )(hist)
    return jnp.reshape(loss, ())


# speedup vs baseline: 1.7616x; 1.7616x over previous
import numpy as np
import jax
import jax.numpy as jnp
from jax import lax
from jax.experimental import pallas as pl
from jax.experimental.pallas import tpu as pltpu
from jax.experimental.pallas import tpu_sc as plsc

LANES = 16
ROWS_PER_UNIT = 192
COLS = 384
PIXELS = 147456.0
VECS_PER_ROW = COLS // LANES
NW = 32

_step = (1.0 - 0.0) / 25
_NODES = np.array([0.0 + _step * n for n in range(25)] + [2.0] * 7,
                  dtype=np.float32)


def _sc_hist_kernel(inp_ref, tgt_ref, nodes_ref, out_ref, buf, nodes_v, hist):
    c = lax.axis_index("c")
    s = lax.axis_index("s")
    wid = s * 2 + c
    tensor = wid // 16
    rem = wid - tensor * 16
    half = rem // 8
    batch = rem - half * 8
    row0 = batch * 384 + half * ROWS_PER_UNIT

    pltpu.sync_copy(nodes_ref, nodes_v)

    @pl.when(tensor == 0)
    def _():
        pltpu.sync_copy(
            inp_ref.at[pl.ds(row0 * COLS, ROWS_PER_UNIT * COLS)], buf)

    @pl.when(tensor == 1)
    def _():
        pltpu.sync_copy(
            tgt_ref.at[pl.ds(row0 * COLS, ROWS_PER_UNIT * COLS)], buf)

    zeros_i = jnp.zeros((LANES,), jnp.int32)
    ones_i = jnp.ones((LANES,), jnp.int32)
    neg_ones = jnp.full((LANES,), -1, jnp.int32)
    v25f = jnp.full((LANES,), 25.0, jnp.float32)
    eps = jnp.full((LANES,), 1e-5, jnp.float32)
    lane = lax.iota(jnp.int32, LANES)

    for k in range(32):
        hist[k, :] = zeros_i

    @plsc.parallel_loop(0, ROWS_PER_UNIT)
    def _loop(r):
        base = r * COLS
        for cvec in range(VECS_PER_ROW):
            x = buf[pl.ds(base + cvec * LANES, LANES)]
            j1 = (x * v25f + eps).astype(jnp.int32)
            lo = plsc.load_gather(nodes_v, [j1])
            j = j1 + jnp.where(x < lo, neg_ones, zeros_i)
            plsc.addupdate_scatter(hist, [j, lane], ones_i)

    pltpu.sync_copy(hist, out_ref.at[wid])


def _tc_loss_kernel(h_ref, o_ref):
    h = h_ref[...].astype(jnp.float32)
    h2 = jnp.sum(h, axis=2)
    cin = h2[0:8] + h2[8:16]
    ctg = h2[16:24] + h2[24:32]
    d = jnp.sum(jnp.abs(cin - ctg), axis=0, keepdims=True)
    a = jnp.sum(cin, axis=0, keepdims=True)
    c = jnp.sum(ctg, axis=0, keepdims=True)
    li = jnp.clip(d - 0.5 * jnp.minimum(a, c), 0.0, PIXELS) / PIXELS
    mask = lax.broadcasted_iota(jnp.int32, (1, 32), 1) < 24
    li = jnp.where(mask, li, 0.0)
    o_ref[...] = jnp.sum(li, axis=(0, 1), keepdims=True) / 24.0


def kernel(_input, _target):
    inp2 = jnp.reshape(_input, (-1,))
    tgt2 = jnp.reshape(_target, (-1,))
    nodes = jnp.asarray(_NODES)

    mesh = plsc.VectorSubcoreMesh(core_axis_name="c", subcore_axis_name="s")
    hist = pl.kernel(
        _sc_hist_kernel,
        out_type=jax.ShapeDtypeStruct((NW, 32, LANES), jnp.int32),
        mesh=mesh,
        scratch_types=[
            pltpu.VMEM((ROWS_PER_UNIT * COLS,), jnp.float32),
            pltpu.VMEM((32,), jnp.float32),
            pltpu.VMEM((32, LANES), jnp.int32),
        ],
        compiler_params=pltpu.CompilerParams(needs_layout_passes=False),
    )(inp2, tgt2, nodes)

    loss = pl.pallas_call(
        _tc_loss_kernel,
        out_shape=jax.ShapeDtypeStruct((1, 1), jnp.float32),
    )(hist)
    return jnp.reshape(loss, ())
